# Initial kernel scaffold; baseline (speedup 1.0000x reference)
import jax
import jax.numpy as jnp
from jax.experimental import pallas as pl


def kernel(x, W, expert_bias, temperature):
    raise NotImplementedError("write your pallas kernel here")


# speedup vs baseline: 2.2212x; 2.2212x over previous
import jax
import jax.numpy as jnp
from jax.experimental import pallas as pl
from jax.experimental.pallas import tpu as pltpu

_TOP_K = 2
_LOAD_BALANCE_ALPHA = 0.01
_Z_LOSS_ALPHA = 0.0001


def _gate_tile(x_ref, wt_ref, bias_ref, temp_ref,
               ts_ref, ti_ref, pf_ref, pp_ref, pz_ref):
    x = x_ref[...]
    logits = jnp.dot(x, wt_ref[...], preferred_element_type=jnp.float32)
    logits = (logits + bias_ref[...]) / temp_ref[0, 0]
    m = jnp.max(logits, axis=-1, keepdims=True)
    ex = jnp.exp(logits - m)
    se = jnp.sum(ex, axis=-1, keepdims=True)
    scores = ex / se
    lse = m + jnp.log(se)
    num_e = scores.shape[-1]
    eidx = jax.lax.broadcasted_iota(jnp.int32, scores.shape, 1)
    v1 = jnp.max(scores, axis=-1, keepdims=True)
    i1 = jnp.min(jnp.where(scores == v1, eidx, num_e), axis=-1, keepdims=True)
    masked = jnp.where(eidx == i1, -1.0, scores)
    v2 = jnp.max(masked, axis=-1, keepdims=True)
    i2 = jnp.min(jnp.where(masked == v2, eidx, num_e), axis=-1, keepdims=True)
    denom = v1 + v2
    ts_ref[...] = jnp.concatenate([v1 / denom, v2 / denom], axis=1)
    ti_ref[...] = jnp.concatenate([i1, i2], axis=1)
    hits = (eidx == i1).astype(jnp.float32) + (eidx == i2).astype(jnp.float32)
    pf_ref[...] = jnp.sum(hits, axis=0, keepdims=True)[None]
    pp_ref[...] = jnp.sum(scores, axis=0, keepdims=True)[None]
    pz_ref[...] = jnp.broadcast_to(jnp.sum(lse * lse), pz_ref.shape)


def kernel(x, W, expert_bias, temperature):
    B, S, H = x.shape
    E = W.shape[0]
    T = B * S
    x_flat = x.reshape(T, H)
    wt = W.T
    bias = expert_bias.reshape(1, E)
    temp = jnp.asarray(temperature, jnp.float32).reshape(1, 1)
    TT = 2048
    G = T // TT
    ts, ti, pf, pp, pz = pl.pallas_call(
        _gate_tile,
        grid=(G,),
        in_specs=[
            pl.BlockSpec((TT, H), lambda i: (i, 0)),
            pl.BlockSpec((H, E), lambda i: (0, 0)),
            pl.BlockSpec((1, E), lambda i: (0, 0)),
            pl.BlockSpec((1, 1), lambda i: (0, 0)),
        ],
        out_specs=[
            pl.BlockSpec((TT, _TOP_K), lambda i: (i, 0)),
            pl.BlockSpec((TT, _TOP_K), lambda i: (i, 0)),
            pl.BlockSpec((1, 1, E), lambda i: (i, 0, 0)),
            pl.BlockSpec((1, 1, E), lambda i: (i, 0, 0)),
            pl.BlockSpec((1, 1, E), lambda i: (i, 0, 0)),
        ],
        out_shape=[
            jax.ShapeDtypeStruct((T, _TOP_K), jnp.float32),
            jax.ShapeDtypeStruct((T, _TOP_K), jnp.int32),
            jax.ShapeDtypeStruct((G, 1, E), jnp.float32),
            jax.ShapeDtypeStruct((G, 1, E), jnp.float32),
            jax.ShapeDtypeStruct((G, 1, E), jnp.float32),
        ],
        compiler_params=pltpu.CompilerParams(
            dimension_semantics=("parallel",)),
    )(x_flat, wt, bias, temp)
    f = jnp.sum(pf[:, 0, :], axis=0) / T
    P = jnp.sum(pp[:, 0, :], axis=0) / T
    z = jnp.sum(pz[:, 0, 0]) / T
    aux = _LOAD_BALANCE_ALPHA * E * jnp.sum(f * P)
    total = aux + _Z_LOSS_ALPHA * z
    return ts, ti, total


# speedup vs baseline: 2.3636x; 1.0641x over previous
import jax
import jax.numpy as jnp
from jax.experimental import pallas as pl
from jax.experimental.pallas import tpu as pltpu

_TOP_K = 2
_LOAD_BALANCE_ALPHA = 0.01
_Z_LOSS_ALPHA = 0.0001


def _gate_tile(x_ref, wt_ref, bias_ref, ts_ref, ti_ref,
               pf_ref, pp_ref, pz_ref):
    logits = jnp.dot(x_ref[...], wt_ref[...],
                     preferred_element_type=jnp.float32)
    lt = logits.T + bias_ref[...]
    c = jnp.max(lt)
    ex = jnp.exp(lt - c)
    se = jnp.sum(ex, axis=0, keepdims=True)
    num_e = ex.shape[0]
    eidx = jax.lax.broadcasted_iota(jnp.int32, ex.shape, 0)
    pack = (jax.lax.bitcast_convert_type(ex, jnp.int32) & ~63) \
        | ((num_e - 1) - eidx)
    r1 = jnp.max(pack, axis=0, keepdims=True)
    m1 = pack == r1
    r2 = jnp.max(jnp.where(m1, 0, pack), axis=0, keepdims=True)
    i1 = (num_e - 1) - (r1 & 63)
    i2 = (num_e - 1) - (r2 & 63)
    v1 = jax.lax.bitcast_convert_type(r1 & ~63, jnp.float32)
    v2 = jax.lax.bitcast_convert_type(r2 & ~63, jnp.float32)
    rden = 1.0 / (v1 + v2)
    ts_ref[...] = jnp.concatenate([v1 * rden, v2 * rden], axis=0).T
    ti_ref[...] = jnp.concatenate([i1, i2], axis=0).T
    hits = m1.astype(jnp.float32) + (pack == r2).astype(jnp.float32)
    lse = c + jnp.log(se)
    pf_ref[...] = jnp.sum(hits, axis=1, keepdims=True).T[None]
    pp_ref[...] = jnp.sum(ex * (1.0 / se), axis=1, keepdims=True).T[None]
    pz_ref[...] = jnp.broadcast_to(jnp.sum(lse * lse), pz_ref.shape)


def kernel(x, W, expert_bias, temperature):
    B, S, H = x.shape
    E = W.shape[0]
    T = B * S
    x_flat = x.reshape(T, H)
    rtemp = 1.0 / jnp.asarray(temperature, jnp.float32)
    wt = W.T * rtemp
    bias = (expert_bias * rtemp).reshape(E, 1)
    TT = 2048
    G = T // TT
    ts, ti, pf, pp, pz = pl.pallas_call(
        _gate_tile,
        grid=(G,),
        in_specs=[
            pl.BlockSpec((TT, H), lambda i: (i, 0)),
            pl.BlockSpec((H, E), lambda i: (0, 0)),
            pl.BlockSpec((E, 1), lambda i: (0, 0)),
        ],
        out_specs=[
            pl.BlockSpec((TT, _TOP_K), lambda i: (i, 0)),
            pl.BlockSpec((TT, _TOP_K), lambda i: (i, 0)),
            pl.BlockSpec((1, 1, E), lambda i: (i, 0, 0)),
            pl.BlockSpec((1, 1, E), lambda i: (i, 0, 0)),
            pl.BlockSpec((1, 1, E), lambda i: (i, 0, 0)),
        ],
        out_shape=[
            jax.ShapeDtypeStruct((T, _TOP_K), jnp.float32),
            jax.ShapeDtypeStruct((T, _TOP_K), jnp.int32),
            jax.ShapeDtypeStruct((G, 1, E), jnp.float32),
            jax.ShapeDtypeStruct((G, 1, E), jnp.float32),
            jax.ShapeDtypeStruct((G, 1, E), jnp.float32),
        ],
        compiler_params=pltpu.CompilerParams(
            dimension_semantics=("parallel",)),
    )(x_flat, wt, bias)
    f = jnp.sum(pf[:, 0, :], axis=0) / T
    P = jnp.sum(pp[:, 0, :], axis=0) / T
    z = jnp.sum(pz[:, 0, 0]) / T
    aux = _LOAD_BALANCE_ALPHA * E * jnp.sum(f * P)
    total = aux + _Z_LOSS_ALPHA * z
    return ts, ti, total


# speedup vs baseline: 3.6421x; 1.5409x over previous
import jax
import jax.numpy as jnp
from jax.experimental import pallas as pl
from jax.experimental.pallas import tpu as pltpu

_TOP_K = 2
_LOAD_BALANCE_ALPHA = 0.01
_Z_LOSS_ALPHA = 0.0001


def _gate_tile(x_ref, wt_ref, bias_ref, rtemp_ref, ts_ref, ti_ref,
               pf_ref, pp_ref, pz_ref):
    logits = jnp.dot(x_ref[...], wt_ref[...],
                     preferred_element_type=jnp.float32)
    lt = (logits.T + bias_ref[...]) * rtemp_ref[0, 0]
    c = jnp.max(lt)
    ex = jnp.exp(lt - c)
    se = jnp.sum(ex, axis=0, keepdims=True)
    num_e = ex.shape[0]
    eidx = jax.lax.broadcasted_iota(jnp.int32, ex.shape, 0)
    pack = (jax.lax.bitcast_convert_type(ex, jnp.int32) & ~63) \
        | ((num_e - 1) - eidx)
    r1 = jnp.max(pack, axis=0, keepdims=True)
    m1 = pack == r1
    r2 = jnp.max(jnp.where(m1, 0, pack), axis=0, keepdims=True)
    i1 = (num_e - 1) - (r1 & 63)
    i2 = (num_e - 1) - (r2 & 63)
    v1 = jax.lax.bitcast_convert_type(r1 & ~63, jnp.float32)
    v2 = jax.lax.bitcast_convert_type(r2 & ~63, jnp.float32)
    rden = 1.0 / (v1 + v2)
    ts_ref[...] = jnp.concatenate([v1 * rden, v2 * rden], axis=0)
    ti_ref[...] = jnp.concatenate([i1, i2], axis=0)
    hits = m1.astype(jnp.float32) + (pack == r2).astype(jnp.float32)
    lse = c + jnp.log(se)
    pf_ref[...] = jnp.sum(hits, axis=1, keepdims=True).T[None]
    pp_ref[...] = jnp.sum(ex * (1.0 / se), axis=1, keepdims=True).T[None]
    pz_ref[...] = jnp.broadcast_to(jnp.sum(lse * lse), pz_ref.shape)


def kernel(x, W, expert_bias, temperature):
    B, S, H = x.shape
    E = W.shape[0]
    T = B * S
    x_flat = x.reshape(T, H)
    rtemp = (1.0 / jnp.asarray(temperature, jnp.float32)).reshape(1, 1)
    wt = W.T
    bias = expert_bias.reshape(E, 1)
    TT = 2048
    G = T // TT
    ts, ti, pf, pp, pz = pl.pallas_call(
        _gate_tile,
        grid=(G,),
        in_specs=[
            pl.BlockSpec((TT, H), lambda i: (i, 0)),
            pl.BlockSpec((H, E), lambda i: (0, 0)),
            pl.BlockSpec((E, 1), lambda i: (0, 0)),
            pl.BlockSpec((1, 1), lambda i: (0, 0)),
        ],
        out_specs=[
            pl.BlockSpec((_TOP_K, TT), lambda i: (0, i)),
            pl.BlockSpec((_TOP_K, TT), lambda i: (0, i)),
            pl.BlockSpec((1, 1, E), lambda i: (i, 0, 0)),
            pl.BlockSpec((1, 1, E), lambda i: (i, 0, 0)),
            pl.BlockSpec((1, 1, E), lambda i: (i, 0, 0)),
        ],
        out_shape=[
            jax.ShapeDtypeStruct((_TOP_K, T), jnp.float32),
            jax.ShapeDtypeStruct((_TOP_K, T), jnp.int32),
            jax.ShapeDtypeStruct((G, 1, E), jnp.float32),
            jax.ShapeDtypeStruct((G, 1, E), jnp.float32),
            jax.ShapeDtypeStruct((G, 1, E), jnp.float32),
        ],
        compiler_params=pltpu.CompilerParams(
            dimension_semantics=("parallel",)),
    )(x_flat, wt, bias, rtemp)
    ts = ts.T
    ti = ti.T
    f = jnp.sum(pf[:, 0, :], axis=0) / T
    P = jnp.sum(pp[:, 0, :], axis=0) / T
    z = jnp.sum(pz[:, 0, 0]) / T
    aux = _LOAD_BALANCE_ALPHA * E * jnp.sum(f * P)
    total = aux + _Z_LOSS_ALPHA * z
    return ts, ti, total


# speedup vs baseline: 3.9960x; 1.0972x over previous
import jax
import jax.numpy as jnp
from jax.experimental import pallas as pl
from jax.experimental.pallas import tpu as pltpu

_TOP_K = 2
_LOAD_BALANCE_ALPHA = 0.01
_Z_LOSS_ALPHA = 0.0001


def _gate_tile(x_ref, wt_ref, bias_ref, rtemp_ref, ts_ref, ti_ref,
               pf_ref, pp_ref, pz_ref):
    logits = jnp.dot(x_ref[...], wt_ref[...],
                     preferred_element_type=jnp.float32)
    lt = (logits.T + bias_ref[...]) * rtemp_ref[0, 0]
    c = jnp.max(lt)
    ex = jnp.exp(lt - c)
    se = jnp.sum(ex, axis=0, keepdims=True)
    num_e = ex.shape[0]
    eidx = jax.lax.broadcasted_iota(jnp.int32, ex.shape, 0)
    pack = (jax.lax.bitcast_convert_type(ex, jnp.int32) & ~63) \
        | ((num_e - 1) - eidx)
    r1 = jnp.max(pack, axis=0, keepdims=True)
    m1 = pack == r1
    r2 = jnp.max(jnp.where(m1, 0, pack), axis=0, keepdims=True)
    i1 = (num_e - 1) - (r1 & 63)
    i2 = (num_e - 1) - (r2 & 63)
    v1 = jax.lax.bitcast_convert_type(r1 & ~63, jnp.float32)
    v2 = jax.lax.bitcast_convert_type(r2 & ~63, jnp.float32)
    rden = 1.0 / (v1 + v2)
    ts_ref[...] = jnp.concatenate([v1 * rden, v2 * rden], axis=0)
    ti_ref[...] = jnp.concatenate([i1, i2], axis=0)
    hits = m1.astype(jnp.float32) + (pack == r2).astype(jnp.float32)
    lse = c + jnp.log(se)
    pf_ref[...] = jnp.sum(hits, axis=1, keepdims=True).T[None]
    pp_ref[...] = jnp.sum(ex * (1.0 / se), axis=1, keepdims=True).T[None]
    pz_ref[...] = jnp.broadcast_to(jnp.sum(lse * lse), pz_ref.shape)


def kernel(x, W, expert_bias, temperature):
    B, S, H = x.shape
    E = W.shape[0]
    T = B * S
    x_flat = x.reshape(T, H)
    rtemp = (1.0 / jnp.asarray(temperature, jnp.float32)).reshape(1, 1)
    wt = W.T
    bias = expert_bias.reshape(E, 1)
    TT = 4096
    G = T // TT
    ts, ti, pf, pp, pz = pl.pallas_call(
        _gate_tile,
        grid=(G,),
        in_specs=[
            pl.BlockSpec((TT, H), lambda i: (i, 0)),
            pl.BlockSpec((H, E), lambda i: (0, 0)),
            pl.BlockSpec((E, 1), lambda i: (0, 0)),
            pl.BlockSpec((1, 1), lambda i: (0, 0)),
        ],
        out_specs=[
            pl.BlockSpec((_TOP_K, TT), lambda i: (0, i)),
            pl.BlockSpec((_TOP_K, TT), lambda i: (0, i)),
            pl.BlockSpec((1, 1, E), lambda i: (i, 0, 0)),
            pl.BlockSpec((1, 1, E), lambda i: (i, 0, 0)),
            pl.BlockSpec((1, 1, E), lambda i: (i, 0, 0)),
        ],
        out_shape=[
            jax.ShapeDtypeStruct((_TOP_K, T), jnp.float32),
            jax.ShapeDtypeStruct((_TOP_K, T), jnp.int32),
            jax.ShapeDtypeStruct((G, 1, E), jnp.float32),
            jax.ShapeDtypeStruct((G, 1, E), jnp.float32),
            jax.ShapeDtypeStruct((G, 1, E), jnp.float32),
        ],
        compiler_params=pltpu.CompilerParams(
            dimension_semantics=("parallel",)),
    )(x_flat, wt, bias, rtemp)
    ts = ts.T
    ti = ti.T
    f = jnp.sum(pf[:, 0, :], axis=0) / T
    P = jnp.sum(pp[:, 0, :], axis=0) / T
    z = jnp.sum(pz[:, 0, 0]) / T
    aux = _LOAD_BALANCE_ALPHA * E * jnp.sum(f * P)
    total = aux + _Z_LOSS_ALPHA * z
    return ts, ti, total
